# baseline (speedup 1.0000x reference)
import functools

import jax
import jax.numpy as jnp
from jax.experimental import pallas as pl
from jax.experimental.pallas import tpu as pltpu

EPS = 1e-5
B = 16
N0 = 2048
PLANES = [32, 64, 128, 256, 512]
NSAMPLE = [8, 16, 16, 16, 16]
STRIDE = [1, 4, 4, 4, 4]
NEG_BIG = -3.0e38
POS_BIG = 3.0e38
MASK_BIG = 1.0e30


def _row_argmin_onehot(d, iota_n, n):
    minv = jnp.min(d, axis=1, keepdims=True)
    cand = jnp.where(d == minv, iota_n, n)
    idx = jnp.min(cand, axis=1, keepdims=True)
    return iota_n == idx


def _fps_stage(px, py, pz, out_ref, dist_ref, m):
    n = px.shape[1]
    iota_n = jax.lax.broadcasted_iota(jnp.int32, (B, n), 1)
    iota_m = jax.lax.broadcasted_iota(jnp.int32, (B, m), 1)
    ps = jnp.concatenate([px, py, pz], axis=0)
    xs0 = px[:, 0:1]
    ys0 = py[:, 0:1]
    zs0 = pz[:, 0:1]
    col0 = iota_m == 0
    zerom = jnp.zeros((B, m), dtype=jnp.float32)
    npx0 = jnp.where(col0, xs0, zerom)
    npy0 = jnp.where(col0, ys0, zerom)
    npz0 = jnp.where(col0, zs0, zerom)
    dist_ref[:, :n] = jnp.full((B, n), 1e10, dtype=jnp.float32)

    def body(i, carry):
        xs, ys, zs, npx, npy, npz = carry
        d = (px - xs) ** 2 + (py - ys) ** 2 + (pz - zs) ** 2
        dist = jnp.minimum(dist_ref[:, :n], d)
        dist_ref[:, :n] = dist
        mx = jnp.max(dist, axis=1, keepdims=True)
        cand = jnp.where(dist == mx, iota_n, n)
        idx = jnp.min(cand, axis=1, keepdims=True)
        oh = iota_n == idx
        oh3 = jnp.concatenate([oh, oh, oh], axis=0)
        sums = jnp.sum(jnp.where(oh3, ps, 0.0), axis=1, keepdims=True)
        xs = sums[0:B]
        ys = sums[B:2 * B]
        zs = sums[2 * B:]
        colmask = iota_m == i
        npx = jnp.where(colmask, xs, npx)
        npy = jnp.where(colmask, ys, npy)
        npz = jnp.where(colmask, zs, npz)
        return xs, ys, zs, npx, npy, npz

    _, _, _, npx, npy, npz = jax.lax.fori_loop(
        1, m, body, (xs0, ys0, zs0, npx0, npy0, npz0))
    out_ref[:, 0, :] = npx
    out_ref[:, 1, :] = npy
    out_ref[:, 2, :] = npz


def _fps_all_kernel(pt_ref, x0_ref, w1_ref, o2_ref, o3_ref, o4_ref, o5_ref,
                    h1_ref, stats1_ref, dist_ref):
    h = jnp.dot(x0_ref[...], w1_ref[...], preferred_element_type=jnp.float32)
    h1_ref[...] = h
    stats1_ref[0, 0:1, :] = jnp.sum(h, axis=0, keepdims=True)
    stats1_ref[0, 1:2, :] = jnp.sum(h * h, axis=0, keepdims=True)
    px = pt_ref[:, 0, :]
    py = pt_ref[:, 1, :]
    pz = pt_ref[:, 2, :]
    _fps_stage(px, py, pz, o2_ref, dist_ref, N0 // 4)
    _fps_stage(o2_ref[:, 0, :], o2_ref[:, 1, :], o2_ref[:, 2, :], o3_ref, dist_ref, N0 // 16)
    _fps_stage(o3_ref[:, 0, :], o3_ref[:, 1, :], o3_ref[:, 2, :], o4_ref, dist_ref, N0 // 64)
    _fps_stage(o4_ref[:, 0, :], o4_ref[:, 1, :], o4_ref[:, 2, :], o5_ref, dist_ref, N0 // 256)


def _apply_prev_bn(sel, stats, gp, bp, count):
    mean = jnp.sum(stats[:, 0, :], axis=0, keepdims=True) / count
    ex2 = jnp.sum(stats[:, 1, :], axis=0, keepdims=True) / count
    var = ex2 - mean * mean
    inv = jax.lax.rsqrt(var + EPS)
    return jnp.maximum(gp * (sel - mean) * inv + bp, 0.0)


def _stage_kernel(p_ref, pt_ref, np_ref, hmax_ref, hmin_ref, stats_ref,
                  gp_ref, bp_ref, w_ref, omax_ref, omin_ref, ostats_ref,
                  dmat_ref, z_ref, oha_ref, ohb_ref,
                  *, n, m, c, dout, nsample, count_prev, pool_prev):
    if pool_prev:
        gp = gp_ref[...]
        sel = jnp.where(gp >= 0.0, hmax_ref[0], hmin_ref[0])
    else:
        sel = hmax_ref[0]
    x = _apply_prev_bn(sel, stats_ref[...], gp_ref[...], bp_ref[...], count_prev)

    hi = jax.lax.Precision.HIGHEST
    wp = w_ref[0:3, :]
    wx = w_ref[3:, :]
    p = p_ref[0]
    npp = np_ref[0]
    z = (jnp.dot(p, wp, preferred_element_type=jnp.float32, precision=hi)
         + jnp.dot(x, wx, preferred_element_type=jnp.float32))
    z_hi = z.astype(jnp.bfloat16).astype(jnp.float32)
    z_ref[:, 0:dout] = z_hi
    z_ref[:, dout:] = z - z_hi
    q = jnp.dot(npp, wp, preferred_element_type=jnp.float32, precision=hi)

    pt = pt_ref[0]
    dmat = ((npp[:, 0:1] - pt[0:1, :]) ** 2
            + (npp[:, 1:2] - pt[1:2, :]) ** 2
            + (npp[:, 2:3] - pt[2:3, :]) ** 2)
    dmat_ref[...] = dmat

    omax_ref[0] = jnp.full((m, dout), NEG_BIG, dtype=jnp.float32)
    omin_ref[0] = jnp.full((m, dout), POS_BIG, dtype=jnp.float32)
    iota_n = jax.lax.broadcasted_iota(jnp.int32, (m, n), 1)

    def select_into(oh_ref):
        d = dmat_ref[...]
        oh = _row_argmin_onehot(d, iota_n, n)
        dmat_ref[...] = jnp.where(oh, MASK_BIG, d)
        oh_ref[...] = oh.astype(jnp.float32)

    def gather_update(oh_ref, acc_s, acc_q):
        gz = jnp.dot(oh_ref[...], z_ref[...], preferred_element_type=jnp.float32)
        hr = gz[:, 0:dout] + gz[:, dout:] - q
        omax_ref[0] = jnp.maximum(omax_ref[0], hr)
        omin_ref[0] = jnp.minimum(omin_ref[0], hr)
        return (acc_s + jnp.sum(hr, axis=0, keepdims=True),
                acc_q + jnp.sum(hr * hr, axis=0, keepdims=True))

    select_into(oha_ref)
    npairs = nsample // 2

    def body(j, carry):
        acc_s, acc_q = carry
        acc_s, acc_q = gather_update(oha_ref, acc_s, acc_q)
        select_into(ohb_ref)
        acc_s, acc_q = gather_update(ohb_ref, acc_s, acc_q)

        @pl.when(j < npairs - 1)
        def _():
            select_into(oha_ref)

        return acc_s, acc_q

    z0 = jnp.zeros((1, dout), dtype=jnp.float32)
    acc_s, acc_q = jax.lax.fori_loop(0, npairs, body, (z0, z0))
    ostats_ref[0, 0:1, :] = acc_s
    ostats_ref[0, 1:2, :] = acc_q


def _batched_stage(nc, mc, dout, x_all, pprev, pt_ref, npf, w_ref, g, bvec,
                   z_ref, d_ref, oh_ref, ga_ref, hx_ref, hn_ref):
    hi = jax.lax.Precision.HIGHEST
    wp = w_ref[0:3, :]
    wx = w_ref[3:, :]
    z = (jnp.dot(pprev, wp, preferred_element_type=jnp.float32, precision=hi)
         + jnp.dot(x_all, wx, preferred_element_type=jnp.float32))
    z_hi = z.astype(jnp.bfloat16).astype(jnp.float32)
    z_ref[:, 0:dout] = z_hi
    z_ref[:, dout:] = z - z_hi
    q = jnp.dot(npf, wp, preferred_element_type=jnp.float32, precision=hi)

    d = None
    for cc in range(3):
        prow = pt_ref[:, cc, :]
        pb = jnp.broadcast_to(prow[:, None, :], (B, mc, nc)).reshape(B * mc, nc)
        t = (npf[:, cc:cc + 1] - pb) ** 2
        d = t if d is None else d + t
    d_ref[...] = d
    hx_ref[...] = jnp.full((B * mc, dout), NEG_BIG, dtype=jnp.float32)
    hn_ref[...] = jnp.full((B * mc, dout), POS_BIG, dtype=jnp.float32)
    iota_n = jax.lax.broadcasted_iota(jnp.int32, (B * mc, nc), 1)

    def round_body(r, carry):
        acc_s, acc_q = carry
        dcur = d_ref[...]
        oh = _row_argmin_onehot(dcur, iota_n, nc)
        d_ref[...] = jnp.where(oh, MASK_BIG, dcur)
        oh_ref[...] = oh.astype(jnp.float32)

        def cloud_body(b, _):
            ohb = oh_ref[pl.ds(b * mc, mc), :]
            zb = z_ref[pl.ds(b * nc, nc), :]
            ga_ref[pl.ds(b * mc, mc), :] = jnp.dot(
                ohb, zb, preferred_element_type=jnp.float32)
            return 0

        jax.lax.fori_loop(0, B, cloud_body, 0)
        g_all = ga_ref[...]
        hr = g_all[:, 0:dout] + g_all[:, dout:] - q
        hx_ref[...] = jnp.maximum(hx_ref[...], hr)
        hn_ref[...] = jnp.minimum(hn_ref[...], hr)
        acc_s = acc_s + jnp.sum(hr, axis=0, keepdims=True)
        acc_q = acc_q + jnp.sum(hr * hr, axis=0, keepdims=True)
        return acc_s, acc_q

    z0 = jnp.zeros((1, dout), dtype=jnp.float32)
    acc_s, acc_q = jax.lax.fori_loop(0, 16, round_body, (z0, z0))
    count = float(B * mc * 16)
    mean = acc_s / count
    var = acc_q / count - mean * mean
    inv = jax.lax.rsqrt(var + EPS)
    sel = jnp.where(g >= 0.0, hx_ref[...], hn_ref[...])
    return jnp.maximum(g * (sel - mean) * inv + bvec, 0.0)


def _stage3_kernel(p2f_ref, pt2_ref, np3f_ref, h2max_ref, h2min_ref,
                   stats2_ref, g2_ref, b2_ref, w3_ref, g3_ref, b3_ref,
                   x3_ref, z3_ref, d3_ref, oh3_ref, ga3_ref, hx3_ref, hn3_ref):
    m2, m3 = N0 // 4, N0 // 16
    g2 = g2_ref[...]
    sel2 = jnp.where(g2 >= 0.0, h2max_ref[...], h2min_ref[...])
    x2 = _apply_prev_bn(sel2, stats2_ref[...], g2, b2_ref[...], float(B * m2 * 16))
    x3_ref[...] = _batched_stage(m2, m3, PLANES[2], x2, p2f_ref[...], pt2_ref,
                                 np3f_ref[...], w3_ref, g3_ref[...], b3_ref[...],
                                 z3_ref, d3_ref, oh3_ref, ga3_ref, hx3_ref, hn3_ref)


def _tail_kernel(x3_ref, np3f_ref, pt3_ref, np4f_ref, pt4_ref, np5f_ref,
                 w4_ref, g4_ref, b4_ref, w5_ref, g5_ref, b5_ref,
                 wo1_ref, bo1_ref, wo2_ref, bo2_ref, out_ref,
                 z4_ref, d4_ref, oh4_ref, ga4_ref, hx4_ref, hn4_ref,
                 z5_ref, d5_ref, oh5_ref, ga5_ref, hx5_ref, hn5_ref):
    m3, m4, m5 = N0 // 16, N0 // 64, N0 // 256
    x4 = _batched_stage(m3, m4, PLANES[3], x3_ref[...], np3f_ref[...], pt3_ref,
                        np4f_ref[...], w4_ref, g4_ref[...], b4_ref[...],
                        z4_ref, d4_ref, oh4_ref, ga4_ref, hx4_ref, hn4_ref)
    x5 = _batched_stage(m4, m5, PLANES[4], x4, np4f_ref[...], pt4_ref,
                        np5f_ref[...], w5_ref, g5_ref[...], b5_ref[...],
                        z5_ref, d5_ref, oh5_ref, ga5_ref, hx5_ref, hn5_ref)
    x = jnp.mean(x5.reshape(B, m5, PLANES[4]), axis=1)
    h = jnp.maximum(jnp.dot(x, wo1_ref[...], preferred_element_type=jnp.float32)
                    + bo1_ref[...], 0.0)
    h = h + x
    out_ref[...] = jnp.dot(h, wo2_ref[...], preferred_element_type=jnp.float32) + bo2_ref[...]


def _full(shape):
    return pl.BlockSpec(shape, lambda b: (0,) * len(shape))


def _per_cloud(shape):
    return pl.BlockSpec((1,) + shape[1:], lambda b: (b,) + (0,) * (len(shape) - 1))


def kernel(coord, feat, offset, w1, g1, b1, w2, g2, b2, w3, g3, b3, w4, g4, b4,
           w5, g5, b5, wo1, bo1, wo2, bo2):
    f32 = jnp.float32
    p_rows = coord.reshape(B, N0, 3)
    pt = jnp.transpose(p_rows, (0, 2, 1))
    x0 = jnp.concatenate([coord, feat.reshape(B * N0, -1)], axis=-1)

    m2, m3, m4, m5 = N0 // 4, N0 // 16, N0 // 64, N0 // 256
    np2, np3, np4, np5, h1, stats1 = pl.pallas_call(
        _fps_all_kernel,
        out_shape=(jax.ShapeDtypeStruct((B, 3, m2), f32),
                   jax.ShapeDtypeStruct((B, 3, m3), f32),
                   jax.ShapeDtypeStruct((B, 3, m4), f32),
                   jax.ShapeDtypeStruct((B, 3, m5), f32),
                   jax.ShapeDtypeStruct((B * N0, PLANES[0]), f32),
                   jax.ShapeDtypeStruct((1, 2, PLANES[0]), f32)),
        scratch_shapes=[pltpu.VMEM((B, N0), f32)],
    )(pt, x0, w1)
    h1 = h1.reshape(B, N0, PLANES[0])

    n, m, c, dout, k = N0, m2, PLANES[0], PLANES[1], NSAMPLE[1]
    np2_rows = jnp.transpose(np2, (0, 2, 1))
    body = functools.partial(
        _stage_kernel, n=n, m=m, c=c, dout=dout, nsample=k,
        count_prev=float(B * N0), pool_prev=False)
    h2max, h2min, stats2 = pl.pallas_call(
        body,
        grid=(B,),
        in_specs=[
            _per_cloud((B, n, 3)),
            _per_cloud((B, 3, n)),
            _per_cloud((B, m, 3)),
            _per_cloud((B, n, c)),
            _per_cloud((B, n, c)),
            _full(stats1.shape),
            _full((1, c)),
            _full((1, c)),
            _full(w2.shape),
        ],
        out_specs=[
            _per_cloud((B, m, dout)),
            _per_cloud((B, m, dout)),
            _per_cloud((B, 2, dout)),
        ],
        out_shape=(jax.ShapeDtypeStruct((B, m, dout), f32),
                   jax.ShapeDtypeStruct((B, m, dout), f32),
                   jax.ShapeDtypeStruct((B, 2, dout), f32)),
        scratch_shapes=[pltpu.VMEM((m, n), f32), pltpu.VMEM((n, 2 * dout), f32),
                        pltpu.VMEM((m, n), f32), pltpu.VMEM((m, n), f32)],
    )(p_rows, pt, np2_rows, h1, h1, stats1, g1.reshape(1, c), b1.reshape(1, c), w2)

    np3_rows = jnp.transpose(np3, (0, 2, 1))
    np4_rows = jnp.transpose(np4, (0, 2, 1))
    np5_rows = jnp.transpose(np5, (0, 2, 1))

    def _stage_scr(mc, nc, dd):
        return [pltpu.VMEM((B * nc, 2 * dd), f32),
                pltpu.VMEM((B * mc, nc), f32),
                pltpu.VMEM((B * mc, nc), f32),
                pltpu.VMEM((B * mc, 2 * dd), f32),
                pltpu.VMEM((B * mc, dd), f32),
                pltpu.VMEM((B * mc, dd), f32)]

    x3 = pl.pallas_call(
        _stage3_kernel,
        out_shape=jax.ShapeDtypeStruct((B * m3, PLANES[2]), f32),
        scratch_shapes=_stage_scr(m3, m2, PLANES[2]),
    )(np2_rows.reshape(B * m2, 3), np2, np3_rows.reshape(B * m3, 3),
      h2max.reshape(B * m2, PLANES[1]), h2min.reshape(B * m2, PLANES[1]), stats2,
      g2.reshape(1, PLANES[1]), b2.reshape(1, PLANES[1]),
      w3, g3.reshape(1, PLANES[2]), b3.reshape(1, PLANES[2]))

    return pl.pallas_call(
        _tail_kernel,
        out_shape=jax.ShapeDtypeStruct((B, 6), f32),
        scratch_shapes=_stage_scr(m4, m3, PLANES[3]) + _stage_scr(m5, m4, PLANES[4]),
    )(x3, np3_rows.reshape(B * m3, 3), np3,
      np4_rows.reshape(B * m4, 3), np4,
      np5_rows.reshape(B * m5, 3),
      w4, g4.reshape(1, PLANES[3]), b4.reshape(1, PLANES[3]),
      w5, g5.reshape(1, PLANES[4]), b5.reshape(1, PLANES[4]),
      wo1, bo1.reshape(1, PLANES[4]), wo2, bo2.reshape(1, 6))


# speedup vs baseline: 1.0216x; 1.0216x over previous
import functools

import jax
import jax.numpy as jnp
from jax.experimental import pallas as pl
from jax.experimental.pallas import tpu as pltpu

EPS = 1e-5
B = 16
N0 = 2048
PLANES = [32, 64, 128, 256, 512]
NSAMPLE = [8, 16, 16, 16, 16]
STRIDE = [1, 4, 4, 4, 4]
NEG_BIG = -3.0e38
POS_BIG = 3.0e38
MASK_BIG = 1.0e30


def _row_argmin_onehot(d, iota_n, n):
    minv = jnp.min(d, axis=1, keepdims=True)
    cand = jnp.where(d == minv, iota_n, n)
    idx = jnp.min(cand, axis=1, keepdims=True)
    return iota_n == idx


def _fps_stage(px, py, pz, out_ref, dist_ref, m):
    n = px.shape[1]
    iota_n = jax.lax.broadcasted_iota(jnp.int32, (B, n), 1)
    iota_m = jax.lax.broadcasted_iota(jnp.int32, (B, m), 1)
    ps = jnp.concatenate([px, py, pz], axis=0)
    xs0 = px[:, 0:1]
    ys0 = py[:, 0:1]
    zs0 = pz[:, 0:1]
    col0 = iota_m == 0
    zerom = jnp.zeros((B, m), dtype=jnp.float32)
    npx0 = jnp.where(col0, xs0, zerom)
    npy0 = jnp.where(col0, ys0, zerom)
    npz0 = jnp.where(col0, zs0, zerom)
    dist_ref[:, :n] = jnp.full((B, n), 1e10, dtype=jnp.float32)

    def body(i, carry):
        xs, ys, zs, npx, npy, npz = carry
        d = (px - xs) ** 2 + (py - ys) ** 2 + (pz - zs) ** 2
        dist = jnp.minimum(dist_ref[:, :n], d)
        dist_ref[:, :n] = dist
        mx = jnp.max(dist, axis=1, keepdims=True)
        cand = jnp.where(dist == mx, iota_n, n)
        idx = jnp.min(cand, axis=1, keepdims=True)
        oh = iota_n == idx
        oh3 = jnp.concatenate([oh, oh, oh], axis=0)
        sums = jnp.sum(jnp.where(oh3, ps, 0.0), axis=1, keepdims=True)
        xs = sums[0:B]
        ys = sums[B:2 * B]
        zs = sums[2 * B:]
        colmask = iota_m == i
        npx = jnp.where(colmask, xs, npx)
        npy = jnp.where(colmask, ys, npy)
        npz = jnp.where(colmask, zs, npz)
        return xs, ys, zs, npx, npy, npz

    _, _, _, npx, npy, npz = jax.lax.fori_loop(
        1, m, body, (xs0, ys0, zs0, npx0, npy0, npz0))
    out_ref[:, 0, :] = npx
    out_ref[:, 1, :] = npy
    out_ref[:, 2, :] = npz


def _fps_all_kernel(pt_ref, x0_ref, w1_ref, o2_ref, o3_ref, o4_ref, o5_ref,
                    h1_ref, stats1_ref, dist_ref):
    h = jnp.dot(x0_ref[...], w1_ref[...], preferred_element_type=jnp.float32)
    h1_ref[...] = h
    stats1_ref[0, 0:1, :] = jnp.sum(h, axis=0, keepdims=True)
    stats1_ref[0, 1:2, :] = jnp.sum(h * h, axis=0, keepdims=True)
    px = pt_ref[:, 0, :]
    py = pt_ref[:, 1, :]
    pz = pt_ref[:, 2, :]
    _fps_stage(px, py, pz, o2_ref, dist_ref, N0 // 4)
    _fps_stage(o2_ref[:, 0, :], o2_ref[:, 1, :], o2_ref[:, 2, :], o3_ref, dist_ref, N0 // 16)
    _fps_stage(o3_ref[:, 0, :], o3_ref[:, 1, :], o3_ref[:, 2, :], o4_ref, dist_ref, N0 // 64)
    _fps_stage(o4_ref[:, 0, :], o4_ref[:, 1, :], o4_ref[:, 2, :], o5_ref, dist_ref, N0 // 256)


def _apply_prev_bn(sel, stats, gp, bp, count):
    mean = jnp.sum(stats[:, 0, :], axis=0, keepdims=True) / count
    ex2 = jnp.sum(stats[:, 1, :], axis=0, keepdims=True) / count
    var = ex2 - mean * mean
    inv = jax.lax.rsqrt(var + EPS)
    return jnp.maximum(gp * (sel - mean) * inv + bp, 0.0)


def _stage_kernel(p_ref, pt_ref, np_ref, hmax_ref, hmin_ref, stats_ref,
                  gp_ref, bp_ref, w_ref, omax_ref, omin_ref, ostats_ref,
                  dmat_ref, z_ref,
                  *, n, m, c, dout, nsample, count_prev, pool_prev):
    if pool_prev:
        gp = gp_ref[...]
        sel = jnp.where(gp >= 0.0, hmax_ref[0], hmin_ref[0])
    else:
        sel = hmax_ref[0]
    x = _apply_prev_bn(sel, stats_ref[...], gp_ref[...], bp_ref[...], count_prev)

    hi = jax.lax.Precision.HIGHEST
    wp = w_ref[0:3, :]
    wx = w_ref[3:, :]
    p = p_ref[0]
    npp = np_ref[0]
    z = (jnp.dot(p, wp, preferred_element_type=jnp.float32, precision=hi)
         + jnp.dot(x, wx, preferred_element_type=jnp.float32))
    z_hi = z.astype(jnp.bfloat16).astype(jnp.float32)
    z_ref[:, 0:dout] = z_hi
    z_ref[:, dout:] = z - z_hi
    q = jnp.dot(npp, wp, preferred_element_type=jnp.float32, precision=hi)

    pt = pt_ref[0]
    dmat = ((npp[:, 0:1] - pt[0:1, :]) ** 2
            + (npp[:, 1:2] - pt[1:2, :]) ** 2
            + (npp[:, 2:3] - pt[2:3, :]) ** 2)
    dmat_ref[...] = dmat

    omax_ref[0] = jnp.full((m, dout), NEG_BIG, dtype=jnp.float32)
    omin_ref[0] = jnp.full((m, dout), POS_BIG, dtype=jnp.float32)
    iota_n = jax.lax.broadcasted_iota(jnp.int32, (m, n), 1)

    def body(r, carry):
        acc_s, acc_q = carry
        d = dmat_ref[...]
        oh = _row_argmin_onehot(d, iota_n, n)
        dmat_ref[...] = jnp.where(oh, MASK_BIG, d)
        ohf = oh.astype(jnp.float32)
        gz = jnp.dot(ohf, z_ref[...], preferred_element_type=jnp.float32)
        hr = gz[:, 0:dout] + gz[:, dout:] - q
        omax_ref[0] = jnp.maximum(omax_ref[0], hr)
        omin_ref[0] = jnp.minimum(omin_ref[0], hr)
        acc_s = acc_s + jnp.sum(hr, axis=0, keepdims=True)
        acc_q = acc_q + jnp.sum(hr * hr, axis=0, keepdims=True)
        return acc_s, acc_q

    z0 = jnp.zeros((1, dout), dtype=jnp.float32)
    acc_s, acc_q = jax.lax.fori_loop(0, nsample, body, (z0, z0))
    ostats_ref[0, 0:1, :] = acc_s
    ostats_ref[0, 1:2, :] = acc_q


def _batched_stage(nc, mc, dout, x_all, pprev, pt_ref, npf, w_ref, g, bvec,
                   z_ref, d_ref, oh_ref, ga_ref, hx_ref, hn_ref):
    hi = jax.lax.Precision.HIGHEST
    wp = w_ref[0:3, :]
    wx = w_ref[3:, :]
    z = (jnp.dot(pprev, wp, preferred_element_type=jnp.float32, precision=hi)
         + jnp.dot(x_all, wx, preferred_element_type=jnp.float32))
    z_hi = z.astype(jnp.bfloat16).astype(jnp.float32)
    z_ref[:, 0:dout] = z_hi
    z_ref[:, dout:] = z - z_hi
    q = jnp.dot(npf, wp, preferred_element_type=jnp.float32, precision=hi)

    d = None
    for cc in range(3):
        prow = pt_ref[:, cc, :]
        pb = jnp.broadcast_to(prow[:, None, :], (B, mc, nc)).reshape(B * mc, nc)
        t = (npf[:, cc:cc + 1] - pb) ** 2
        d = t if d is None else d + t
    d_ref[...] = d
    hx_ref[...] = jnp.full((B * mc, dout), NEG_BIG, dtype=jnp.float32)
    hn_ref[...] = jnp.full((B * mc, dout), POS_BIG, dtype=jnp.float32)
    iota_n = jax.lax.broadcasted_iota(jnp.int32, (B * mc, nc), 1)

    def round_body(r, carry):
        acc_s, acc_q = carry
        dcur = d_ref[...]
        oh = _row_argmin_onehot(dcur, iota_n, nc)
        d_ref[...] = jnp.where(oh, MASK_BIG, dcur)
        oh_ref[...] = oh.astype(jnp.float32)

        def cloud_body(b, _):
            ohb = oh_ref[pl.ds(b * mc, mc), :]
            zb = z_ref[pl.ds(b * nc, nc), :]
            ga_ref[pl.ds(b * mc, mc), :] = jnp.dot(
                ohb, zb, preferred_element_type=jnp.float32)
            return 0

        jax.lax.fori_loop(0, B, cloud_body, 0)
        g_all = ga_ref[...]
        hr = g_all[:, 0:dout] + g_all[:, dout:] - q
        hx_ref[...] = jnp.maximum(hx_ref[...], hr)
        hn_ref[...] = jnp.minimum(hn_ref[...], hr)
        acc_s = acc_s + jnp.sum(hr, axis=0, keepdims=True)
        acc_q = acc_q + jnp.sum(hr * hr, axis=0, keepdims=True)
        return acc_s, acc_q

    z0 = jnp.zeros((1, dout), dtype=jnp.float32)
    acc_s, acc_q = jax.lax.fori_loop(0, 16, round_body, (z0, z0))
    count = float(B * mc * 16)
    mean = acc_s / count
    var = acc_q / count - mean * mean
    inv = jax.lax.rsqrt(var + EPS)
    sel = jnp.where(g >= 0.0, hx_ref[...], hn_ref[...])
    return jnp.maximum(g * (sel - mean) * inv + bvec, 0.0)


def _stage3_kernel(p2f_ref, pt2_ref, np3f_ref, h2max_ref, h2min_ref,
                   stats2_ref, g2_ref, b2_ref, w3_ref, g3_ref, b3_ref,
                   x3_ref, z3_ref, d3_ref, oh3_ref, ga3_ref, hx3_ref, hn3_ref):
    m2, m3 = N0 // 4, N0 // 16
    g2 = g2_ref[...]
    sel2 = jnp.where(g2 >= 0.0, h2max_ref[...], h2min_ref[...])
    x2 = _apply_prev_bn(sel2, stats2_ref[...], g2, b2_ref[...], float(B * m2 * 16))
    x3_ref[...] = _batched_stage(m2, m3, PLANES[2], x2, p2f_ref[...], pt2_ref,
                                 np3f_ref[...], w3_ref, g3_ref[...], b3_ref[...],
                                 z3_ref, d3_ref, oh3_ref, ga3_ref, hx3_ref, hn3_ref)


def _tail_kernel(x3_ref, np3f_ref, pt3_ref, np4f_ref, pt4_ref, np5f_ref,
                 w4_ref, g4_ref, b4_ref, w5_ref, g5_ref, b5_ref,
                 wo1_ref, bo1_ref, wo2_ref, bo2_ref, out_ref,
                 z4_ref, d4_ref, oh4_ref, ga4_ref, hx4_ref, hn4_ref,
                 z5_ref, d5_ref, oh5_ref, ga5_ref, hx5_ref, hn5_ref):
    m3, m4, m5 = N0 // 16, N0 // 64, N0 // 256
    x4 = _batched_stage(m3, m4, PLANES[3], x3_ref[...], np3f_ref[...], pt3_ref,
                        np4f_ref[...], w4_ref, g4_ref[...], b4_ref[...],
                        z4_ref, d4_ref, oh4_ref, ga4_ref, hx4_ref, hn4_ref)
    x5 = _batched_stage(m4, m5, PLANES[4], x4, np4f_ref[...], pt4_ref,
                        np5f_ref[...], w5_ref, g5_ref[...], b5_ref[...],
                        z5_ref, d5_ref, oh5_ref, ga5_ref, hx5_ref, hn5_ref)
    x = jnp.mean(x5.reshape(B, m5, PLANES[4]), axis=1)
    h = jnp.maximum(jnp.dot(x, wo1_ref[...], preferred_element_type=jnp.float32)
                    + bo1_ref[...], 0.0)
    h = h + x
    out_ref[...] = jnp.dot(h, wo2_ref[...], preferred_element_type=jnp.float32) + bo2_ref[...]


def _full(shape):
    return pl.BlockSpec(shape, lambda b: (0,) * len(shape))


def _per_cloud(shape):
    return pl.BlockSpec((1,) + shape[1:], lambda b: (b,) + (0,) * (len(shape) - 1))


def kernel(coord, feat, offset, w1, g1, b1, w2, g2, b2, w3, g3, b3, w4, g4, b4,
           w5, g5, b5, wo1, bo1, wo2, bo2):
    f32 = jnp.float32
    p_rows = coord.reshape(B, N0, 3)
    pt = jnp.transpose(p_rows, (0, 2, 1))
    x0 = jnp.concatenate([coord, feat.reshape(B * N0, -1)], axis=-1)

    m2, m3, m4, m5 = N0 // 4, N0 // 16, N0 // 64, N0 // 256
    np2, np3, np4, np5, h1, stats1 = pl.pallas_call(
        _fps_all_kernel,
        out_shape=(jax.ShapeDtypeStruct((B, 3, m2), f32),
                   jax.ShapeDtypeStruct((B, 3, m3), f32),
                   jax.ShapeDtypeStruct((B, 3, m4), f32),
                   jax.ShapeDtypeStruct((B, 3, m5), f32),
                   jax.ShapeDtypeStruct((B * N0, PLANES[0]), f32),
                   jax.ShapeDtypeStruct((1, 2, PLANES[0]), f32)),
        scratch_shapes=[pltpu.VMEM((B, N0), f32)],
    )(pt, x0, w1)
    h1 = h1.reshape(B, N0, PLANES[0])

    n, m, c, dout, k = N0, m2, PLANES[0], PLANES[1], NSAMPLE[1]
    np2_rows = jnp.transpose(np2, (0, 2, 1))
    body = functools.partial(
        _stage_kernel, n=n, m=m, c=c, dout=dout, nsample=k,
        count_prev=float(B * N0), pool_prev=False)
    h2max, h2min, stats2 = pl.pallas_call(
        body,
        grid=(B,),
        in_specs=[
            _per_cloud((B, n, 3)),
            _per_cloud((B, 3, n)),
            _per_cloud((B, m, 3)),
            _per_cloud((B, n, c)),
            _per_cloud((B, n, c)),
            _full(stats1.shape),
            _full((1, c)),
            _full((1, c)),
            _full(w2.shape),
        ],
        out_specs=[
            _per_cloud((B, m, dout)),
            _per_cloud((B, m, dout)),
            _per_cloud((B, 2, dout)),
        ],
        out_shape=(jax.ShapeDtypeStruct((B, m, dout), f32),
                   jax.ShapeDtypeStruct((B, m, dout), f32),
                   jax.ShapeDtypeStruct((B, 2, dout), f32)),
        scratch_shapes=[pltpu.VMEM((m, n), f32), pltpu.VMEM((n, 2 * dout), f32)],
    )(p_rows, pt, np2_rows, h1, h1, stats1, g1.reshape(1, c), b1.reshape(1, c), w2)

    np3_rows = jnp.transpose(np3, (0, 2, 1))
    np4_rows = jnp.transpose(np4, (0, 2, 1))
    np5_rows = jnp.transpose(np5, (0, 2, 1))

    def _stage_scr(mc, nc, dd):
        return [pltpu.VMEM((B * nc, 2 * dd), f32),
                pltpu.VMEM((B * mc, nc), f32),
                pltpu.VMEM((B * mc, nc), f32),
                pltpu.VMEM((B * mc, 2 * dd), f32),
                pltpu.VMEM((B * mc, dd), f32),
                pltpu.VMEM((B * mc, dd), f32)]

    x3 = pl.pallas_call(
        _stage3_kernel,
        out_shape=jax.ShapeDtypeStruct((B * m3, PLANES[2]), f32),
        scratch_shapes=_stage_scr(m3, m2, PLANES[2]),
    )(np2_rows.reshape(B * m2, 3), np2, np3_rows.reshape(B * m3, 3),
      h2max.reshape(B * m2, PLANES[1]), h2min.reshape(B * m2, PLANES[1]), stats2,
      g2.reshape(1, PLANES[1]), b2.reshape(1, PLANES[1]),
      w3, g3.reshape(1, PLANES[2]), b3.reshape(1, PLANES[2]))

    return pl.pallas_call(
        _tail_kernel,
        out_shape=jax.ShapeDtypeStruct((B, 6), f32),
        scratch_shapes=_stage_scr(m4, m3, PLANES[3]) + _stage_scr(m5, m4, PLANES[4]),
    )(x3, np3_rows.reshape(B * m3, 3), np3,
      np4_rows.reshape(B * m4, 3), np4,
      np5_rows.reshape(B * m5, 3),
      w4, g4.reshape(1, PLANES[3]), b4.reshape(1, PLANES[3]),
      w5, g5.reshape(1, PLANES[4]), b5.reshape(1, PLANES[4]),
      wo1, bo1.reshape(1, PLANES[4]), wo2, bo2.reshape(1, 6))
